# baseline (speedup 1.0000x reference)
import functools

import jax
import jax.numpy as jnp
from jax import lax
from jax.experimental import pallas as pl
from jax.experimental.pallas import tpu as pltpu
from jax.experimental.pallas import tpu_sc as plsc

NC = 2
NS = 16
NW = NC * NS
L = 16
D = 64

VB = 1 << 20
SC_CHUNK = 128

CH = 256
SC_ROWS = 86 * 32 * CH
TC_BLK = 8192


def _sc_histogram(idx, n_idx):
    per_tile = n_idx // NW
    nstream = per_tile // SC_CHUNK
    slice_per_tile = VB // NS
    mesh = plsc.VectorSubcoreMesh(core_axis_name="c", subcore_axis_name="s")

    @functools.partial(
        pl.kernel,
        out_type=jax.ShapeDtypeStruct((NC * VB,), jnp.float32),
        mesh=mesh,
        scratch_types=[
            pltpu.VMEM((per_tile,), jnp.int32),
            pltpu.VMEM((SC_CHUNK,), jnp.float32),
            pltpu.VMEM((slice_per_tile // 4,), jnp.float32),
            pltpu.VMEM_SHARED((VB,), jnp.float32),
            pltpu.SemaphoreType.DMA,
        ],
    )
    def k(idx_hbm, out_hbm, idx_v, ones_v, zero_v, hist_sp, sem):
        core = lax.axis_index("c")
        sub = lax.axis_index("s")
        base = (core * NS + sub) * per_tile
        pltpu.sync_copy(idx_hbm.at[pl.ds(base, per_tile)], idx_v)

        def fill_ones(kk, _):
            ones_v[pl.ds(kk * L, L)] = jnp.ones((L,), jnp.float32)
            return 0

        lax.fori_loop(0, SC_CHUNK // L, fill_ones, 0)

        def fill_zero(kk, _):
            zero_v[pl.ds(kk * L, L)] = jnp.zeros((L,), jnp.float32)
            return 0

        qtr = slice_per_tile // 4
        lax.fori_loop(0, qtr // L, fill_zero, 0, unroll=8)

        for q in range(4):
            pltpu.sync_copy(
                zero_v, hist_sp.at[pl.ds(sub * slice_per_tile + q * qtr, qtr)]
            )
        plsc.subcore_barrier()

        def fire(cc, _):
            pltpu.async_copy(
                ones_v,
                hist_sp.at[idx_v.at[pl.ds(cc * SC_CHUNK, SC_CHUNK)]],
                sem,
                add=True,
            )
            return 0

        lax.fori_loop(0, nstream, fire, 0)

        def drain(cc, _):
            pltpu.make_async_copy(
                ones_v,
                hist_sp.at[idx_v.at[pl.ds(0, SC_CHUNK)]],
                sem,
            ).wait()
            return 0

        lax.fori_loop(0, nstream, drain, 0)

        plsc.subcore_barrier()
        pltpu.sync_copy(
            hist_sp.at[pl.ds(sub * slice_per_tile, slice_per_tile)],
            out_hbm.at[pl.ds(core * VB + sub * slice_per_tile, slice_per_tile)],
        )

    return k(idx)


def _sc_matvec(counts, table):
    per_tile_rows = SC_ROWS // NW
    nch = per_tile_rows // CH
    mesh = plsc.VectorSubcoreMesh(core_axis_name="c", subcore_axis_name="s")

    @functools.partial(
        pl.kernel,
        out_type=jax.ShapeDtypeStruct((NW, D), jnp.float32),
        mesh=mesh,
        scratch_types=[
            pltpu.VMEM((2, CH, D), jnp.float32),
            pltpu.VMEM((2, 2, CH), jnp.float32),
            pltpu.VMEM((D,), jnp.float32),
            pltpu.SemaphoreType.DMA((2,)),
            pltpu.SemaphoreType.DMA((2,)),
        ],
        compiler_params=pltpu.CompilerParams(use_tc_tiling_on_sc=True),
    )
    def k(cnt_hbm, tab_hbm, out_hbm, tbuf, cbuf, acc_v, tsems, csems):
        core = lax.axis_index("c")
        sub = lax.axis_index("s")
        wid = core * NS + sub
        base = wid * per_tile_rows

        def start(c, slot):
            r0 = base + c * CH
            pltpu.make_async_copy(
                tab_hbm.at[pl.ds(r0, CH), :], tbuf.at[slot], tsems.at[slot]
            ).start()
            pltpu.make_async_copy(
                cnt_hbm.at[pl.ds(r0, CH)], cbuf.at[slot, 0], csems.at[slot]
            ).start()
            pltpu.make_async_copy(
                cnt_hbm.at[pl.ds(VB + r0, CH)], cbuf.at[slot, 1], csems.at[slot]
            ).start()

        def wait(slot):
            pltpu.make_async_copy(
                tab_hbm.at[pl.ds(0, CH), :], tbuf.at[slot], tsems.at[slot]
            ).wait()
            pltpu.make_async_copy(
                cnt_hbm.at[pl.ds(0, CH)], cbuf.at[slot, 0], csems.at[slot]
            ).wait()
            pltpu.make_async_copy(
                cnt_hbm.at[pl.ds(0, CH)], cbuf.at[slot, 1], csems.at[slot]
            ).wait()

        for b in range(2):
            start(b, b)

        def group_body(g, a, slot):
            a0, a1, a2, a3 = a
            c16 = (
                cbuf[slot, 0, pl.ds(g * L, L)]
                + cbuf[slot, 1, pl.ds(g * L, L)]
            )
            for kk in range(L):
                cv = jnp.full((L,), c16[kk], jnp.float32)
                i = g * L + kk
                a0 = a0 + tbuf[slot, i, pl.ds(0, L)] * cv
                a1 = a1 + tbuf[slot, i, pl.ds(L, L)] * cv
                a2 = a2 + tbuf[slot, i, pl.ds(2 * L, L)] * cv
                a3 = a3 + tbuf[slot, i, pl.ds(3 * L, L)] * cv
            return (a0, a1, a2, a3)

        def outer_body(co, carry):
            for b in range(2):
                c = co * 2 + b
                wait(b)
                carry = lax.fori_loop(
                    0, CH // L, functools.partial(group_body, slot=b), carry
                )

                @pl.when(c + 2 < nch)
                def _():
                    start(c + 2, b)

            return carry

        z = jnp.zeros((L,), jnp.float32)
        a0, a1, a2, a3 = lax.fori_loop(0, nch // 2, outer_body, (z, z, z, z))
        acc_v[pl.ds(0, L)] = a0
        acc_v[pl.ds(L, L)] = a1
        acc_v[pl.ds(2 * L, L)] = a2
        acc_v[pl.ds(3 * L, L)] = a3
        pltpu.sync_copy(acc_v, out_hbm.at[wid])

    return k(counts, table)


def _tc_matvec(counts, table):
    v_rows = table.shape[0]
    d = table.shape[1]
    blk0 = SC_ROWS // TC_BLK
    nblk = (v_rows - SC_ROWS + TC_BLK - 1) // TC_BLK

    def body(ca_ref, cb_ref, t_ref, o_ref, acc_ref):
        i = pl.program_id(0)

        @pl.when(i == 0)
        def _():
            acc_ref[...] = jnp.zeros_like(acc_ref)

        c = (ca_ref[...] + cb_ref[...]).reshape(1, TC_BLK)
        acc_ref[...] += lax.dot_general(
            c, t_ref[...], (((1,), (0,)), ((), ())),
            preferred_element_type=jnp.float32,
        )

        @pl.when(i == nblk - 1)
        def _():
            o_ref[...] = acc_ref[...]

    return pl.pallas_call(
        body,
        grid=(nblk,),
        in_specs=[
            pl.BlockSpec((TC_BLK,), lambda i: (blk0 + i,)),
            pl.BlockSpec((TC_BLK,), lambda i: (VB // TC_BLK + blk0 + i,)),
            pl.BlockSpec((TC_BLK, d), lambda i: (blk0 + i, 0)),
        ],
        out_specs=pl.BlockSpec((1, d), lambda i: (0, 0)),
        out_shape=jax.ShapeDtypeStruct((1, d), jnp.float32),
        scratch_shapes=[pltpu.VMEM((1, d), jnp.float32)],
    )(counts, counts, table)


def _tc_combine(sc_part, tc_part, w, b, n_rows, n_idx):
    def body(p_ref, q_ref, w_ref, b_ref, o_ref):
        s = (jnp.sum(p_ref[...], axis=0, keepdims=True) + q_ref[...]) * (
            1.0 / n_idx
        )
        y = lax.dot_general(
            s, w_ref[...], (((1,), (1,)), ((), ())),
            preferred_element_type=jnp.float32,
        ) + b_ref[...][None, :]
        y = jnp.where(y >= 0, y, 0.01 * y)
        o_ref[...] = jnp.broadcast_to(y, o_ref.shape)

    return pl.pallas_call(
        body,
        out_shape=jax.ShapeDtypeStruct((n_rows, w.shape[0]), jnp.float32),
    )(sc_part, tc_part, w, b)


@jax.jit
def kernel(x, mp_neighbors, item_table, neigh_w, neigh_b, mp):
    flat_idx = mp_neighbors.reshape(-1)
    counts = _sc_histogram(flat_idx, flat_idx.shape[0])
    sc_part = _sc_matvec(counts, item_table)
    tc_part = _tc_matvec(counts, item_table)
    return _tc_combine(
        sc_part, tc_part, neigh_w, neigh_b, x.shape[0], flat_idx.shape[0]
    )
